# baseline (speedup 1.0000x reference)
import functools
import math

import jax
import jax.numpy as jnp
from jax import lax
from jax.experimental import pallas as pl
from jax.experimental.pallas import tpu as pltpu
from jax.experimental.pallas import tpu_sc as plsc

_EMB = 128
_LANES = 16
_IDX_CHUNK = 128


def _sc_workers():
  try:
    info = plsc.get_sparse_core_info()
    return info.num_cores, info.num_subcores
  except Exception:
    return 2, 16


@functools.cache
def _build(B, S):
  N = B * S
  NC, NS = _sc_workers()
  NW = NC * NS
  assert N % (NW * _IDX_CHUNK) == 0
  rows_w = N // NW
  assert S % rows_w == 0
  n_chunks = rows_w // _IDX_CHUNK
  scale = jnp.float32(math.sqrt(_EMB))
  mesh = plsc.VectorSubcoreMesh(core_axis_name="c", subcore_axis_name="s")

  @functools.partial(
      pl.kernel,
      mesh=mesh,
      out_type=jax.ShapeDtypeStruct((N, _EMB), jnp.float32),
      scratch_types=[
          pltpu.VMEM((n_chunks, _IDX_CHUNK), jnp.int32),
          pltpu.VMEM((n_chunks, _IDX_CHUNK), jnp.int32),
          pltpu.VMEM((rows_w, _EMB), jnp.float32),
          pltpu.VMEM((rows_w, _EMB), jnp.float32),
          pltpu.VMEM((rows_w, _EMB), jnp.float32),
          pltpu.SemaphoreType.DMA,
          pltpu.SemaphoreType.DMA,
          pltpu.SemaphoreType.DMA,
      ],
  )
  def fused(ids_hbm, tt_ids_hbm, word_hbm, tt_hbm, pos_hbm, out_hbm,
            idx_v, tt_idx_v, word_v, tt_v, pos_v, sem_w, sem_t, sem_p):
    wid = lax.axis_index("s") * NC + lax.axis_index("c")
    base = wid * rows_w
    pltpu.sync_copy(ids_hbm.at[pl.ds(wid * n_chunks, n_chunks)], idx_v)
    pltpu.sync_copy(tt_ids_hbm.at[pl.ds(wid * n_chunks, n_chunks)], tt_idx_v)
    copies = []
    for j in range(n_chunks):
      copies.append(pltpu.async_copy(
          word_hbm.at[idx_v.at[j]],
          word_v.at[pl.ds(j * _IDX_CHUNK, _IDX_CHUNK)], sem_w))
      copies.append(pltpu.async_copy(
          tt_hbm.at[tt_idx_v.at[j]],
          tt_v.at[pl.ds(j * _IDX_CHUNK, _IDX_CHUNK)], sem_t))
    pos_base = pl.multiple_of(jnp.bitwise_and(base, S - 1), 8)
    copies.append(pltpu.async_copy(
        pos_hbm.at[pl.ds(pos_base, rows_w)], pos_v, sem_p))
    for c in copies:
      c.wait()

    def body(r, carry):
      for c0 in range(_EMB // _LANES):
        sl = pl.ds(c0 * _LANES, _LANES)
        word_v[r, sl] = word_v[r, sl] * scale + tt_v[r, sl] + pos_v[r, sl]
      return carry

    lax.fori_loop(0, rows_w, body, 0)
    pltpu.sync_copy(word_v, out_hbm.at[pl.ds(base, rows_w)])

  return fused


def kernel(input_ids, seq_length, token_type_ids, word_embeddings,
           token_type_table, position_embeddings):
  del seq_length
  B, S = input_ids.shape
  fused = _build(B, S)
  out = fused(input_ids.reshape(-1, _IDX_CHUNK),
              token_type_ids.reshape(-1, _IDX_CHUNK),
              word_embeddings, token_type_table, position_embeddings)
  return out.reshape(B, S, _EMB)
